# baseline (speedup 1.0000x reference)
import functools

import jax
import jax.numpy as jnp
from jax import lax
from jax.experimental import pallas as pl
from jax.experimental.pallas import tpu as pltpu
from jax.experimental.pallas import tpu_sc as plsc

N = 10000
E = 320000
F = 128
G = 128
C = 10

NC = 2
NS = 16
ET = E // NS
K = 128
SECL = 2000
NSEC = ET // SECL
MAXCH = (SECL + K - 1) // K + 2
HALF = 5120
RPT = HALF // NS
ACC_R = HALF + 8
NP = NC * HALF


def _sc_aggregate_body(x_hbm, src_hbm, dst_hbm, out_hbm,
                       raw_src, raw_dst, csrc, cdst, dstbuf,
                       rows0, rows1, agg, sem0, sem1):
    c = lax.axis_index("c")
    s = lax.axis_index("s")
    lo = c * HALF

    def _zrow(r, _):
        for l in range(F // 16):
            rows0[r, pl.ds(l * 16, 16)] = jnp.zeros((16,), jnp.float32)
        return 0
    lax.fori_loop(0, K, _zrow, 0)

    def _zcopy(j, _):
        pltpu.sync_copy(rows0.at[pl.ds(0, 64)], agg.at[pl.ds(s * RPT + j * 64, 64)])
        return 0
    lax.fori_loop(0, RPT // 64, _zcopy, 0)

    plsc.subcore_barrier()

    lanes = lax.iota(jnp.int32, 16)
    pad_src = (lanes & 7) * 512
    pad_dst = HALF + (lanes & 7)

    for sec in range(NSEC):
        pltpu.sync_copy(src_hbm.at[s * NSEC + sec, 0], raw_src)
        pltpu.sync_copy(dst_hbm.at[s * NSEC + sec, 0], raw_dst)

        def _pre(o, _):
            csrc[pl.ds(o * 16, 16)] = pad_src
            cdst[pl.ds(o * 16, 16)] = pad_dst
            return 0
        lax.fori_loop(0, (MAXCH * K) // 16, _pre, 0)

        def _comp(o, cnt):
            s16 = raw_src[pl.ds(o * 16, 16)]
            d16 = raw_dst[pl.ds(o * 16, 16)]
            m = (d16 >= lo) & (d16 < lo + HALF)
            mi = jnp.where(m, jnp.ones((16,), jnp.int32), jnp.zeros((16,), jnp.int32))
            pos = cnt + plsc.cumsum(mi) - 1
            plsc.store_scatter(csrc, [pos], s16, mask=m)
            plsc.store_scatter(cdst, [pos], d16 - lo, mask=m)
            return cnt + jnp.sum(mi)
        cnt = lax.fori_loop(0, SECL // 16, _comp, 0)
        nch = (cnt + K - 1) // K

        pltpu.async_copy(x_hbm.at[csrc.at[pl.ds(0, K)]], rows0, sem0)
        npairs = nch // 2

        def _pair(j, _):
            a = 2 * j
            pltpu.async_copy(x_hbm.at[csrc.at[pl.ds((a + 1) * K, K)]],
                             rows1, sem1)
            for l in range(K // 16):
                dstbuf[pl.ds(l * 16, 16)] = cdst[pl.ds(a * K + l * 16, 16)]
            pltpu.make_async_copy(x_hbm.at[csrc.at[pl.ds(a * K, K)]],
                                  rows0, sem0).wait()
            pltpu.sync_copy(rows0, agg.at[dstbuf], add=True)
            pltpu.async_copy(x_hbm.at[csrc.at[pl.ds((a + 2) * K, K)]],
                             rows0, sem0)
            for l in range(K // 16):
                dstbuf[pl.ds(l * 16, 16)] = cdst[pl.ds((a + 1) * K + l * 16, 16)]
            pltpu.make_async_copy(x_hbm.at[csrc.at[pl.ds((a + 1) * K, K)]],
                                  rows1, sem1).wait()
            pltpu.sync_copy(rows1, agg.at[dstbuf], add=True)
            return 0
        lax.fori_loop(0, npairs, _pair, 0)

        last = nch - 1

        @pl.when(nch % 2 == 1)
        def _odd():
            pltpu.make_async_copy(x_hbm.at[csrc.at[pl.ds(last * K, K)]],
                                  rows0, sem0).wait()
            for l in range(K // 16):
                dstbuf[pl.ds(l * 16, 16)] = cdst[pl.ds(last * K + l * 16, 16)]
            pltpu.sync_copy(rows0, agg.at[dstbuf], add=True)

        @pl.when(nch % 2 == 0)
        def _even():
            pltpu.make_async_copy(x_hbm.at[csrc.at[pl.ds(0, K)]],
                                  rows0, sem0).wait()

    plsc.subcore_barrier()

    pltpu.sync_copy(agg.at[pl.ds(s * RPT, RPT)], out_hbm.at[c, pl.ds(s * RPT, RPT)])


@functools.cache
def _get_sc_aggregate():
    mesh = plsc.VectorSubcoreMesh(
        core_axis_name="c", subcore_axis_name="s",
        num_cores=NC, num_subcores=NS)
    return pl.kernel(
        _sc_aggregate_body,
        out_type=jax.ShapeDtypeStruct((NC, HALF, F), jnp.float32),
        mesh=mesh,
        compiler_params=pltpu.CompilerParams(needs_layout_passes=False),
        scratch_types=[
            pltpu.VMEM((SECL,), jnp.int32),
            pltpu.VMEM((SECL,), jnp.int32),
            pltpu.VMEM((MAXCH * K,), jnp.int32),
            pltpu.VMEM((MAXCH * K,), jnp.int32),
            pltpu.VMEM((K,), jnp.int32),
            pltpu.VMEM((K, F), jnp.float32),
            pltpu.VMEM((K, F), jnp.float32),
            pltpu.VMEM_SHARED((ACC_R, F), jnp.float32),
            pltpu.SemaphoreType.DMA,
            pltpu.SemaphoreType.DMA,
        ],
    )


def _sc_aggregate(x, src, dst):
    return _get_sc_aggregate()(x, src, dst).reshape(NP, F)


_RB = 1000


def _conv_body(p_ref, x_ref, wrel_ref, b_ref, wroot_ref, o_ref):
    acc = jnp.dot(p_ref[...], wrel_ref[...], preferred_element_type=jnp.float32)
    acc += jnp.dot(x_ref[...], wroot_ref[...], preferred_element_type=jnp.float32)
    o_ref[...] = jnp.maximum(acc + b_ref[...], 0.0)


def _conv_res_body(p_ref, x_ref, wrel_ref, b_ref, wroot_ref, r_ref, o_ref):
    acc = jnp.dot(p_ref[...], wrel_ref[...], preferred_element_type=jnp.float32)
    acc += jnp.dot(x_ref[...], wroot_ref[...], preferred_element_type=jnp.float32)
    o_ref[...] = jnp.maximum(acc + b_ref[...], 0.0) + r_ref[...]


def _conv_tc(p, x, W_rel, b, W_root, res=None):
    grid = (N // _RB,)
    in_specs = [
        pl.BlockSpec((_RB, F), lambda i: (i, 0)),
        pl.BlockSpec((_RB, F), lambda i: (i, 0)),
        pl.BlockSpec((F, F), lambda i: (0, 0)),
        pl.BlockSpec((1, F), lambda i: (0, 0)),
        pl.BlockSpec((F, F), lambda i: (0, 0)),
    ]
    args = [p, x, W_rel, b.reshape(1, F), W_root]
    body = _conv_body
    if res is not None:
        in_specs.append(pl.BlockSpec((_RB, F), lambda i: (i, 0)))
        args.append(res)
        body = _conv_res_body
    return pl.pallas_call(
        body,
        grid=grid,
        in_specs=in_specs,
        out_specs=pl.BlockSpec((_RB, F), lambda i: (i, 0)),
        out_shape=jax.ShapeDtypeStruct((N, F), jnp.float32),
    )(*args)


def _pool_head_body(h_ref, b_ref, l1a_ref, l1b_ref, l1bias_ref,
                    l2w_ref, l2b_ref, l3w_ref, l3b_ref, o_ref,
                    sum_acc, max_acc, cnt_acc):
    i = pl.program_id(0)
    nb = pl.num_programs(0)

    @pl.when(i == 0)
    def _init():
        sum_acc[...] = jnp.zeros_like(sum_acc)
        max_acc[...] = jnp.full_like(max_acc, -jnp.inf)
        cnt_acc[...] = jnp.zeros_like(cnt_acc)

    h = h_ref[...]
    bids = b_ref[...]
    oh = (bids == lax.broadcasted_iota(jnp.int32, (_RB, G), 1))
    ohf = oh.astype(jnp.float32)
    sum_acc[...] += lax.dot_general(ohf, h, (((0,), (0,)), ((), ())),
                                    preferred_element_type=jnp.float32)
    cnt_acc[...] += jnp.sum(ohf, axis=0, keepdims=True)

    lo = b_ref[0, 0]
    hi = b_ref[_RB - 1, 0]

    def _seg_max(g, _):
        eq = (bids == g)
        m = jnp.max(jnp.where(eq, h, -jnp.inf), axis=0, keepdims=True)
        max_acc[pl.ds(g, 1), :] = jnp.maximum(max_acc[pl.ds(g, 1), :], m)
        return 0
    lax.fori_loop(lo, hi + 1, _seg_max, 0)

    @pl.when(i == nb - 1)
    def _head():
        cnt = cnt_acc[0, :]
        gap = sum_acc[...] / jnp.maximum(cnt, 1.0)[:, None]
        gmp = jnp.where(cnt[:, None] > 0, max_acc[...], 0.0)
        z = jnp.maximum(
            jnp.dot(gmp, l1a_ref[...], preferred_element_type=jnp.float32)
            + jnp.dot(gap, l1b_ref[...], preferred_element_type=jnp.float32)
            + l1bias_ref[...], 0.0)
        z = jnp.maximum(
            jnp.dot(z, l2w_ref[...], preferred_element_type=jnp.float32)
            + l2b_ref[...], 0.0)
        logits = jnp.dot(z, l3w_ref[...], preferred_element_type=jnp.float32) \
            + l3b_ref[...]
        m = jnp.max(logits, axis=1, keepdims=True)
        lse = m + jnp.log(jnp.sum(jnp.exp(logits - m), axis=1, keepdims=True))
        o_ref[...] = logits - lse


def _pool_head(h, batch2, L1_W, L1_b, L2_W, L2_b, L3_W, L3_b):
    grid = (N // _RB,)
    return pl.pallas_call(
        _pool_head_body,
        grid=grid,
        in_specs=[
            pl.BlockSpec((_RB, F), lambda i: (i, 0)),
            pl.BlockSpec((_RB, 1), lambda i: (i, 0)),
            pl.BlockSpec((F, F), lambda i: (0, 0)),
            pl.BlockSpec((F, F), lambda i: (0, 0)),
            pl.BlockSpec((1, F), lambda i: (0, 0)),
            pl.BlockSpec((F, 64), lambda i: (0, 0)),
            pl.BlockSpec((1, 64), lambda i: (0, 0)),
            pl.BlockSpec((64, C), lambda i: (0, 0)),
            pl.BlockSpec((1, C), lambda i: (0, 0)),
        ],
        out_specs=pl.BlockSpec((G, C), lambda i: (0, 0)),
        out_shape=jax.ShapeDtypeStruct((G, C), jnp.float32),
        scratch_shapes=[
            pltpu.VMEM((G, F), jnp.float32),
            pltpu.VMEM((G, F), jnp.float32),
            pltpu.VMEM((1, G), jnp.float32),
        ],
    )(h, batch2, L1_W[:F], L1_W[F:], L1_b.reshape(1, F),
      L2_W, L2_b.reshape(1, 64), L3_W, L3_b.reshape(1, C))


def kernel(x, edge_index, batch,
           W1_rel, b1_rel, W1_root,
           W2_rel, b2_rel, W2_root,
           W3_rel, b3_rel, W3_root,
           L1_W, L1_b, L2_W, L2_b, L3_W, L3_b):
    src = edge_index[0].reshape(NS * NSEC, 1, SECL)
    dst = edge_index[1].reshape(NS * NSEC, 1, SECL)
    batch2 = batch.reshape(N, 1)

    p = _sc_aggregate(x, src, dst)
    x1 = _conv_tc(p, x, W1_rel, b1_rel, W1_root)
    p = _sc_aggregate(x1, src, dst)
    x2 = _conv_tc(p, x1, W2_rel, b2_rel, W2_root)
    p = _sc_aggregate(x2, src, dst)
    x3r = _conv_tc(p, x2, W3_rel, b3_rel, W3_root, res=x1)
    p = _sc_aggregate(x3r, src, dst)
    x4 = _conv_tc(p, x3r, W3_rel, b3_rel, W3_root)
    return _pool_head(x4, batch2, L1_W, L1_b, L2_W, L2_b, L3_W, L3_b)


# speedup vs baseline: 2.0170x; 2.0170x over previous
import functools

import jax
import jax.numpy as jnp
from jax import lax
from jax.experimental import pallas as pl
from jax.experimental.pallas import tpu as pltpu
from jax.experimental.pallas import tpu_sc as plsc

N = 10000
E = 320000
F = 128
G = 128
C = 10

NC = 2
NS = 16
ET = E // NS
K = 80
SECL = 4000
NSEC = ET // SECL
MAXCH = (SECL + K - 1) // K + 2
HALF = 5120
RPT = HALF // NS
ACC_R = HALF + 8
NP = NC * HALF


def _sc_aggregate_body(x_hbm, src_hbm, dst_hbm, out_hbm,
                       raw_src, raw_dst, csrc, cdst, dstbuf,
                       rows0, rows1, agg, sem0, sem1):
    c = lax.axis_index("c")
    s = lax.axis_index("s")
    lo = c * HALF

    def _zrow(r, _):
        for l in range(F // 16):
            rows0[r, pl.ds(l * 16, 16)] = jnp.zeros((16,), jnp.float32)
        return 0
    lax.fori_loop(0, K, _zrow, 0)

    def _zcopy(j, _):
        pltpu.sync_copy(rows0.at[pl.ds(0, 64)], agg.at[pl.ds(s * RPT + j * 64, 64)])
        return 0
    lax.fori_loop(0, RPT // 64, _zcopy, 0)

    plsc.subcore_barrier()

    lanes = lax.iota(jnp.int32, 16)
    pad_src = (lanes & 7) * 512
    pad_dst = HALF + (lanes & 7)

    for sec in range(NSEC):
        pltpu.sync_copy(src_hbm.at[s * NSEC + sec, 0], raw_src)
        pltpu.sync_copy(dst_hbm.at[s * NSEC + sec, 0], raw_dst)

        def _pre(o, _):
            csrc[pl.ds(o * 16, 16)] = pad_src
            cdst[pl.ds(o * 16, 16)] = pad_dst
            return 0
        lax.fori_loop(0, (MAXCH * K) // 16, _pre, 0)

        def _comp(o, cnt):
            s16 = raw_src[pl.ds(o * 16, 16)]
            d16 = raw_dst[pl.ds(o * 16, 16)]
            m = (d16 >= lo) & (d16 < lo + HALF)
            mi = jnp.where(m, jnp.ones((16,), jnp.int32), jnp.zeros((16,), jnp.int32))
            pos = cnt + plsc.cumsum(mi) - 1
            plsc.store_scatter(csrc, [pos], s16, mask=m)
            plsc.store_scatter(cdst, [pos], d16 - lo, mask=m)
            return cnt + jnp.sum(mi)
        cnt = lax.fori_loop(0, SECL // 16, _comp, 0)
        nch = (cnt + K - 1) // K

        pltpu.async_copy(x_hbm.at[csrc.at[pl.ds(0, K)]], rows0, sem0)
        npairs = nch // 2

        def _pair(j, _):
            a = 2 * j
            pltpu.async_copy(x_hbm.at[csrc.at[pl.ds((a + 1) * K, K)]],
                             rows1, sem1)
            for l in range(K // 16):
                dstbuf[pl.ds(l * 16, 16)] = cdst[pl.ds(a * K + l * 16, 16)]
            pltpu.make_async_copy(x_hbm.at[csrc.at[pl.ds(a * K, K)]],
                                  rows0, sem0).wait()
            pltpu.sync_copy(rows0, agg.at[dstbuf], add=True)
            pltpu.async_copy(x_hbm.at[csrc.at[pl.ds((a + 2) * K, K)]],
                             rows0, sem0)
            for l in range(K // 16):
                dstbuf[pl.ds(l * 16, 16)] = cdst[pl.ds((a + 1) * K + l * 16, 16)]
            pltpu.make_async_copy(x_hbm.at[csrc.at[pl.ds((a + 1) * K, K)]],
                                  rows1, sem1).wait()
            pltpu.sync_copy(rows1, agg.at[dstbuf], add=True)
            return 0
        lax.fori_loop(0, npairs, _pair, 0)

        last = nch - 1

        @pl.when(nch % 2 == 1)
        def _odd():
            pltpu.make_async_copy(x_hbm.at[csrc.at[pl.ds(last * K, K)]],
                                  rows0, sem0).wait()
            for l in range(K // 16):
                dstbuf[pl.ds(l * 16, 16)] = cdst[pl.ds(last * K + l * 16, 16)]
            pltpu.sync_copy(rows0, agg.at[dstbuf], add=True)

        @pl.when(nch % 2 == 0)
        def _even():
            pltpu.make_async_copy(x_hbm.at[csrc.at[pl.ds(0, K)]],
                                  rows0, sem0).wait()

    plsc.subcore_barrier()

    pltpu.sync_copy(agg.at[pl.ds(s * RPT, RPT)], out_hbm.at[c, pl.ds(s * RPT, RPT)])


@functools.cache
def _get_sc_aggregate():
    mesh = plsc.VectorSubcoreMesh(
        core_axis_name="c", subcore_axis_name="s",
        num_cores=NC, num_subcores=NS)
    return pl.kernel(
        _sc_aggregate_body,
        out_type=jax.ShapeDtypeStruct((NC, HALF, F), jnp.float32),
        mesh=mesh,
        compiler_params=pltpu.CompilerParams(needs_layout_passes=False),
        scratch_types=[
            pltpu.VMEM((SECL,), jnp.int32),
            pltpu.VMEM((SECL,), jnp.int32),
            pltpu.VMEM((MAXCH * K,), jnp.int32),
            pltpu.VMEM((MAXCH * K,), jnp.int32),
            pltpu.VMEM((K,), jnp.int32),
            pltpu.VMEM((K, F), jnp.float32),
            pltpu.VMEM((K, F), jnp.float32),
            pltpu.VMEM_SHARED((ACC_R, F), jnp.float32),
            pltpu.SemaphoreType.DMA,
            pltpu.SemaphoreType.DMA,
        ],
    )


def _sc_aggregate(x, src, dst):
    return _get_sc_aggregate()(x, src, dst).reshape(NP, F)


_RB = 1000


def _conv_body(p_ref, x_ref, wrel_ref, b_ref, wroot_ref, o_ref):
    acc = jnp.dot(p_ref[...], wrel_ref[...], preferred_element_type=jnp.float32)
    acc += jnp.dot(x_ref[...], wroot_ref[...], preferred_element_type=jnp.float32)
    o_ref[...] = jnp.maximum(acc + b_ref[...], 0.0)


def _conv_res_body(p_ref, x_ref, wrel_ref, b_ref, wroot_ref, r_ref, o_ref):
    acc = jnp.dot(p_ref[...], wrel_ref[...], preferred_element_type=jnp.float32)
    acc += jnp.dot(x_ref[...], wroot_ref[...], preferred_element_type=jnp.float32)
    o_ref[...] = jnp.maximum(acc + b_ref[...], 0.0) + r_ref[...]


def _conv_tc(p, x, W_rel, b, W_root, res=None):
    grid = (N // _RB,)
    in_specs = [
        pl.BlockSpec((_RB, F), lambda i: (i, 0)),
        pl.BlockSpec((_RB, F), lambda i: (i, 0)),
        pl.BlockSpec((F, F), lambda i: (0, 0)),
        pl.BlockSpec((1, F), lambda i: (0, 0)),
        pl.BlockSpec((F, F), lambda i: (0, 0)),
    ]
    args = [p, x, W_rel, b.reshape(1, F), W_root]
    body = _conv_body
    if res is not None:
        in_specs.append(pl.BlockSpec((_RB, F), lambda i: (i, 0)))
        args.append(res)
        body = _conv_res_body
    return pl.pallas_call(
        body,
        grid=grid,
        in_specs=in_specs,
        out_specs=pl.BlockSpec((_RB, F), lambda i: (i, 0)),
        out_shape=jax.ShapeDtypeStruct((N, F), jnp.float32),
    )(*args)


def _pool_head_body(h_ref, b_ref, l1a_ref, l1b_ref, l1bias_ref,
                    l2w_ref, l2b_ref, l3w_ref, l3b_ref, o_ref,
                    sum_acc, max_acc, cnt_acc):
    i = pl.program_id(0)
    nb = pl.num_programs(0)

    @pl.when(i == 0)
    def _init():
        sum_acc[...] = jnp.zeros_like(sum_acc)
        max_acc[...] = jnp.full_like(max_acc, -jnp.inf)
        cnt_acc[...] = jnp.zeros_like(cnt_acc)

    h = h_ref[...]
    bids = b_ref[...]
    oh = (bids == lax.broadcasted_iota(jnp.int32, (_RB, G), 1))
    ohf = oh.astype(jnp.float32)
    sum_acc[...] += lax.dot_general(ohf, h, (((0,), (0,)), ((), ())),
                                    preferred_element_type=jnp.float32)
    cnt_acc[...] += jnp.sum(ohf, axis=0, keepdims=True)

    lo = b_ref[0, 0]
    hi = b_ref[_RB - 1, 0]

    def _seg_max(g, _):
        eq = (bids == g)
        m = jnp.max(jnp.where(eq, h, -jnp.inf), axis=0, keepdims=True)
        max_acc[pl.ds(g, 1), :] = jnp.maximum(max_acc[pl.ds(g, 1), :], m)
        return 0
    lax.fori_loop(lo, hi + 1, _seg_max, 0)

    @pl.when(i == nb - 1)
    def _head():
        cnt = cnt_acc[0, :]
        gap = sum_acc[...] / jnp.maximum(cnt, 1.0)[:, None]
        gmp = jnp.where(cnt[:, None] > 0, max_acc[...], 0.0)
        z = jnp.maximum(
            jnp.dot(gmp, l1a_ref[...], preferred_element_type=jnp.float32)
            + jnp.dot(gap, l1b_ref[...], preferred_element_type=jnp.float32)
            + l1bias_ref[...], 0.0)
        z = jnp.maximum(
            jnp.dot(z, l2w_ref[...], preferred_element_type=jnp.float32)
            + l2b_ref[...], 0.0)
        logits = jnp.dot(z, l3w_ref[...], preferred_element_type=jnp.float32) \
            + l3b_ref[...]
        m = jnp.max(logits, axis=1, keepdims=True)
        lse = m + jnp.log(jnp.sum(jnp.exp(logits - m), axis=1, keepdims=True))
        o_ref[...] = logits - lse


def _pool_head(h, batch2, L1_W, L1_b, L2_W, L2_b, L3_W, L3_b):
    grid = (N // _RB,)
    return pl.pallas_call(
        _pool_head_body,
        grid=grid,
        in_specs=[
            pl.BlockSpec((_RB, F), lambda i: (i, 0)),
            pl.BlockSpec((_RB, 1), lambda i: (i, 0)),
            pl.BlockSpec((F, F), lambda i: (0, 0)),
            pl.BlockSpec((F, F), lambda i: (0, 0)),
            pl.BlockSpec((1, F), lambda i: (0, 0)),
            pl.BlockSpec((F, 64), lambda i: (0, 0)),
            pl.BlockSpec((1, 64), lambda i: (0, 0)),
            pl.BlockSpec((64, C), lambda i: (0, 0)),
            pl.BlockSpec((1, C), lambda i: (0, 0)),
        ],
        out_specs=pl.BlockSpec((G, C), lambda i: (0, 0)),
        out_shape=jax.ShapeDtypeStruct((G, C), jnp.float32),
        scratch_shapes=[
            pltpu.VMEM((G, F), jnp.float32),
            pltpu.VMEM((G, F), jnp.float32),
            pltpu.VMEM((1, G), jnp.float32),
        ],
    )(h, batch2, L1_W[:F], L1_W[F:], L1_b.reshape(1, F),
      L2_W, L2_b.reshape(1, 64), L3_W, L3_b.reshape(1, C))


def kernel(x, edge_index, batch,
           W1_rel, b1_rel, W1_root,
           W2_rel, b2_rel, W2_root,
           W3_rel, b3_rel, W3_root,
           L1_W, L1_b, L2_W, L2_b, L3_W, L3_b):
    src = edge_index[0].reshape(NS * NSEC, 1, SECL)
    dst = edge_index[1].reshape(NS * NSEC, 1, SECL)
    batch2 = batch.reshape(N, 1)

    p = _sc_aggregate(x, src, dst)
    x1 = _conv_tc(p, x, W1_rel, b1_rel, W1_root)
    p = _sc_aggregate(x1, src, dst)
    x2 = _conv_tc(p, x1, W2_rel, b2_rel, W2_root)
    p = _sc_aggregate(x2, src, dst)
    x3r = _conv_tc(p, x2, W3_rel, b3_rel, W3_root, res=x1)
    p = _sc_aggregate(x3r, src, dst)
    x4 = _conv_tc(p, x3r, W3_rel, b3_rel, W3_root)
    return _pool_head(x4, batch2, L1_W, L1_b, L2_W, L2_b, L3_W, L3_b)
